# baseline (speedup 1.0000x reference)
import functools

import jax
import jax.numpy as jnp
from jax.experimental import pallas as pl
from jax.experimental.pallas import tpu as pltpu

N = 16384
NC = 1000
R = 256
G = N // R


def _stats_kernel(logit_ref, lab_ref, hard_ref, soft_ref, wrong_ref, f1s_ref):
    x = logit_ref[...]
    lab = lab_ref[...]
    col = jax.lax.broadcasted_iota(jnp.int32, (R, NC), 1)
    onehot = col == lab
    masked = jnp.where(onehot, x, 0.0)
    ones = jnp.ones((NC, 128), jnp.float32)
    l1 = jax.lax.dot_general(
        masked, ones, (((1,), (0,)), ((), ())),
        precision=jax.lax.Precision.HIGHEST,
        preferred_element_type=jnp.float32)[:, 0:1]
    mmax = jnp.max(jnp.where(onehot, -jnp.inf, x), axis=1, keepdims=True)
    m1 = jnp.maximum(l1, mmax)
    ex = jnp.exp(x - m1)
    se = jax.lax.dot_general(
        ex, ones, (((1,), (0,)), ((), ())),
        precision=jax.lax.Precision.HIGHEST,
        preferred_element_type=jnp.float32)[:, 0:1]
    lse = m1 + jnp.log(se)

    tie = l1 == mmax
    f1s_ref[...] = (l1 > mmax).astype(jnp.float32)

    @pl.when(jnp.any(tie))
    def _():
        ismax = x == m1
        am = jnp.min(jnp.where(ismax, col, NC), axis=1, keepdims=True)
        f1s_ref[...] = (am == lab).astype(jnp.float32)

    f1 = f1s_ref[...] > 0.5
    hard = jnp.maximum(1.0 - l1 + mmax, 0.0)
    soft = jnp.maximum(1.0 - l1 + jnp.where(f1, mmax, lse), 0.0)
    hard_ref[...] = hard
    soft_ref[...] = soft
    wrong_ref[...] = (~f1).astype(jnp.float32)


def _select_kernel(hard_ref, soft_ref, wrong_ref, out_ref):
    hard = hard_ref[...]
    soft = soft_ref[...]
    nf = jnp.float32(N)
    E = jnp.sum(wrong_ref[...])
    C = nf + E
    bits = jnp.maximum(jax.lax.bitcast_convert_type(hard, jnp.int32), 0)
    r0 = jax.lax.broadcasted_iota(jnp.int32, (128, 128), 0)
    c0 = jax.lax.broadcasted_iota(jnp.int32, (128, 128), 1)
    idx = r0 * 128 + c0

    def cnt_of(b):
        return jnp.sum(jnp.where(bits <= b, 1.0, 0.0))

    def cnt_sum_of(b):
        mask = bits <= b
        return (jnp.sum(jnp.where(mask, 1.0, 0.0)),
                jnp.sum(jnp.where(mask, hard, 0.0)))

    def body1(k, count):
        cand = count + jax.lax.shift_left(jnp.int32(1), 30 - k)
        m, s = cnt_sum_of(cand - 1)
        return jnp.where(s + m - 1.0 <= C, cand, count)

    F = jax.lax.fori_loop(0, 31, body1, jnp.int32(0))
    bstar = F - 1
    m_lo, s_lo = cnt_sum_of(bstar)

    gt_mask = bits > bstar
    v_next = jnp.min(jnp.where(gt_mask, hard, jnp.inf))
    bits_next = jnp.min(jnp.where(gt_mask, bits, jnp.int32(2147483647)))
    c_next = jnp.sum(jnp.where(bits == bits_next, 1.0, 0.0))

    rhs = (C + 1.0 - s_lo + m_lo * v_next) / (v_next + 1.0)
    ns0 = jnp.clip(jnp.floor(rhs), m_lo, m_lo + c_next)
    ns0 = jnp.where(m_lo >= nf, nf, ns0)

    total = jnp.sum(hard)
    ext = ns0 - m_lo
    ls_at = s_lo + jnp.where(ext > 0.0, ext * v_next, 0.0)
    ls_at = jnp.where(ns0 == 0.0, total, ls_at)
    upbound = (ls_at <= C - ns0).astype(jnp.float32)
    ns_f = jnp.minimum(ns0 + upbound, nf)

    def body2(k, count):
        cand = count + jax.lax.shift_left(jnp.int32(1), 30 - k)
        return jnp.where(cnt_of(cand - 1) < ns_f, cand, count)

    B2 = jax.lax.fori_loop(0, 31, body2, jnp.int32(0))
    below = bits < B2
    cnt_less = jnp.sum(jnp.where(below, 1.0, 0.0))
    r = ns_f - cnt_less
    S1 = jnp.sum(jnp.where(below, soft, 0.0))
    group = bits == B2

    def body3(k, count):
        cand = count + jax.lax.shift_left(jnp.int32(1), 13 - k)
        c = jnp.sum(jnp.where(group & (idx <= cand - 1), 1.0, 0.0))
        return jnp.where(c < r, cand, count)

    I = jax.lax.fori_loop(0, 14, body3, jnp.int32(0))
    S2 = jnp.sum(jnp.where(group & (idx <= I), soft, 0.0))
    S2 = jnp.where(r > 0.0, S2, 0.0)
    out_ref[...] = jnp.full((1, 1), (S1 + S2) / ns_f, jnp.float32)


@jax.jit
def kernel(logit, labels):
    lab3 = labels.astype(jnp.int32).reshape(G, R, 1)
    hard, soft, wrong = pl.pallas_call(
        _stats_kernel,
        grid=(G,),
        in_specs=[
            pl.BlockSpec((R, NC), lambda i: (i, 0)),
            pl.BlockSpec((None, R, 1), lambda i: (i, 0, 0)),
        ],
        out_specs=[
            pl.BlockSpec((R, 1), lambda i: (i, 0)),
            pl.BlockSpec((R, 1), lambda i: (i, 0)),
            pl.BlockSpec((R, 1), lambda i: (i, 0)),
        ],
        out_shape=[
            jax.ShapeDtypeStruct((N, 1), jnp.float32),
            jax.ShapeDtypeStruct((N, 1), jnp.float32),
            jax.ShapeDtypeStruct((N, 1), jnp.float32),
        ],
        scratch_shapes=[pltpu.VMEM((R, 1), jnp.float32)],
        compiler_params=pltpu.CompilerParams(
            dimension_semantics=("parallel",)),
    )(logit, lab3)

    h2 = hard.reshape(128, 128)
    s2 = soft.reshape(128, 128)
    w2 = wrong.reshape(128, 128)
    out = pl.pallas_call(
        _select_kernel,
        out_shape=jax.ShapeDtypeStruct((1, 1), jnp.float32),
    )(h2, s2, w2)
    return out.reshape(())


# speedup vs baseline: 2.0630x; 2.0630x over previous
import functools

import jax
import jax.numpy as jnp
from jax.experimental import pallas as pl
from jax.experimental.pallas import tpu as pltpu

N = 16384
NC = 1000
R = 256
G = N // R


def _stats_kernel(logit_ref, lab_ref, hard_ref, soft_ref, wrong_ref, f1s_ref):
    x = logit_ref[...]
    lab = lab_ref[...]
    col = jax.lax.broadcasted_iota(jnp.int32, (R, NC), 1)
    onehot = col == lab
    l1 = jnp.sum(jnp.where(onehot, x, 0.0), axis=1, keepdims=True)
    mmax = jnp.max(jnp.where(onehot, -jnp.inf, x), axis=1, keepdims=True)
    m1 = jnp.maximum(l1, mmax)
    se = jnp.sum(jnp.exp(x - m1), axis=1, keepdims=True)
    lse = m1 + jnp.log(se)

    tie = l1 == mmax
    f1s_ref[...] = (l1 > mmax).astype(jnp.float32)

    @pl.when(jnp.any(tie))
    def _():
        ismax = x == m1
        am = jnp.min(jnp.where(ismax, col, NC), axis=1, keepdims=True)
        f1s_ref[...] = (am == lab).astype(jnp.float32)

    f1 = f1s_ref[...] > 0.5
    hard = jnp.maximum(1.0 - l1 + mmax, 0.0)
    soft = jnp.maximum(1.0 - l1 + jnp.where(f1, mmax, lse), 0.0)
    hard_ref[...] = hard
    soft_ref[...] = soft
    wrong_ref[...] = (~f1).astype(jnp.float32)


def _select_kernel(hard_ref, soft_ref, wrong_ref, out_ref):
    hard = hard_ref[...]
    soft = soft_ref[...]
    nf = jnp.float32(N)
    E = jnp.sum(wrong_ref[...])
    C = nf + E
    bits = jnp.maximum(jax.lax.bitcast_convert_type(hard, jnp.int32), 0)
    r0 = jax.lax.broadcasted_iota(jnp.int32, (128, 128), 0)
    c0 = jax.lax.broadcasted_iota(jnp.int32, (128, 128), 1)
    idx = r0 * 128 + c0

    def cnt_of(b):
        return jnp.sum(jnp.where(bits <= b, 1.0, 0.0))

    def cnt_sum_of(b):
        mask = bits <= b
        return (jnp.sum(jnp.where(mask, 1.0, 0.0)),
                jnp.sum(jnp.where(mask, hard, 0.0)))

    def body1(k, count):
        cand = count + jax.lax.shift_left(jnp.int32(1), 30 - k)
        m, s = cnt_sum_of(cand - 1)
        return jnp.where(s + m - 1.0 <= C, cand, count)

    F = jax.lax.fori_loop(0, 31, body1, jnp.int32(0))
    bstar = F - 1
    m_lo, s_lo = cnt_sum_of(bstar)

    gt_mask = bits > bstar
    v_next = jnp.min(jnp.where(gt_mask, hard, jnp.inf))
    bits_next = jnp.min(jnp.where(gt_mask, bits, jnp.int32(2147483647)))
    c_next = jnp.sum(jnp.where(bits == bits_next, 1.0, 0.0))

    rhs = (C + 1.0 - s_lo + m_lo * v_next) / (v_next + 1.0)
    ns0 = jnp.clip(jnp.floor(rhs), m_lo, m_lo + c_next)
    ns0 = jnp.where(m_lo >= nf, nf, ns0)

    total = jnp.sum(hard)
    ext = ns0 - m_lo
    ls_at = s_lo + jnp.where(ext > 0.0, ext * v_next, 0.0)
    ls_at = jnp.where(ns0 == 0.0, total, ls_at)
    upbound = (ls_at <= C - ns0).astype(jnp.float32)
    ns_f = jnp.minimum(ns0 + upbound, nf)

    def body2(k, count):
        cand = count + jax.lax.shift_left(jnp.int32(1), 30 - k)
        return jnp.where(cnt_of(cand - 1) < ns_f, cand, count)

    B2 = jax.lax.fori_loop(0, 31, body2, jnp.int32(0))
    below = bits < B2
    cnt_less = jnp.sum(jnp.where(below, 1.0, 0.0))
    r = ns_f - cnt_less
    S1 = jnp.sum(jnp.where(below, soft, 0.0))
    group = bits == B2

    def body3(k, count):
        cand = count + jax.lax.shift_left(jnp.int32(1), 13 - k)
        c = jnp.sum(jnp.where(group & (idx <= cand - 1), 1.0, 0.0))
        return jnp.where(c < r, cand, count)

    I = jax.lax.fori_loop(0, 14, body3, jnp.int32(0))
    S2 = jnp.sum(jnp.where(group & (idx <= I), soft, 0.0))
    S2 = jnp.where(r > 0.0, S2, 0.0)
    out_ref[...] = jnp.full((1, 1), (S1 + S2) / ns_f, jnp.float32)


@jax.jit
def kernel(logit, labels):
    lab3 = labels.astype(jnp.int32).reshape(G, R, 1)
    hard, soft, wrong = pl.pallas_call(
        _stats_kernel,
        grid=(G,),
        in_specs=[
            pl.BlockSpec((R, NC), lambda i: (i, 0)),
            pl.BlockSpec((None, R, 1), lambda i: (i, 0, 0)),
        ],
        out_specs=[
            pl.BlockSpec((R, 1), lambda i: (i, 0)),
            pl.BlockSpec((R, 1), lambda i: (i, 0)),
            pl.BlockSpec((R, 1), lambda i: (i, 0)),
        ],
        out_shape=[
            jax.ShapeDtypeStruct((N, 1), jnp.float32),
            jax.ShapeDtypeStruct((N, 1), jnp.float32),
            jax.ShapeDtypeStruct((N, 1), jnp.float32),
        ],
        scratch_shapes=[pltpu.VMEM((R, 1), jnp.float32)],
        compiler_params=pltpu.CompilerParams(
            dimension_semantics=("parallel",)),
    )(logit, lab3)

    h2 = hard.reshape(128, 128)
    s2 = soft.reshape(128, 128)
    w2 = wrong.reshape(128, 128)
    out = pl.pallas_call(
        _select_kernel,
        out_shape=jax.ShapeDtypeStruct((1, 1), jnp.float32),
    )(h2, s2, w2)
    return out.reshape(())
